# baseline (speedup 1.0000x reference)
import jax
import jax.numpy as jnp
import numpy as np
from jax import lax
from jax.experimental import pallas as pl
from jax.experimental.pallas import tpu as pltpu

B = 128
V = 100000
BCA = 16384
NBA = (V + BCA - 1) // BCA
BCW = 32768
NBW = (V + BCW - 1) // BCW


def _threefry2x32(k0, k1, x0, x1):
    rotations = ((13, 15, 26, 6), (17, 29, 16, 24))
    ks = (np.uint32(k0), np.uint32(k1),
          np.uint32(k0) ^ np.uint32(k1) ^ np.uint32(0x1BD11BDA))
    x0 = (x0 + ks[0]).astype(np.uint32)
    x1 = (x1 + ks[1]).astype(np.uint32)
    for i in range(5):
        for r in rotations[i % 2]:
            x0 = (x0 + x1).astype(np.uint32)
            x1 = ((x1 << np.uint32(r)) | (x1 >> np.uint32(32 - r))).astype(np.uint32)
            x1 = x1 ^ x0
        x0 = (x0 + ks[(i + 1) % 3]).astype(np.uint32)
        x1 = (x1 + ks[(i + 2) % 3] + np.uint32(i + 1)).astype(np.uint32)
    return x0, x1


def _uniform_const():
    idx = np.arange(B * V, dtype=np.uint64)
    b0, b1 = _threefry2x32(0, 42,
                           (idx >> np.uint64(32)).astype(np.uint32),
                           (idx & np.uint64(0xFFFFFFFF)).astype(np.uint32))
    bits = b0 ^ b1
    fl = ((bits >> np.uint32(9)) | np.uint32(0x3F800000)).view(np.float32)
    fl = fl - np.float32(1.0)
    tiny = np.float32(np.finfo(np.float32).tiny)
    u = np.maximum(tiny, fl * (np.float32(1.0) - tiny) + tiny)
    return u.reshape(B, V)


_U = _uniform_const()


def _argmax_body(logits_ref, u_ref, idx_ref, best_ref, bidx_ref):
    j = pl.program_id(0)

    @pl.when(j == 0)
    def _():
        best_ref[...] = jnp.full((B, 1), -jnp.inf, jnp.float32)
        bidx_ref[...] = jnp.zeros((B, 1), jnp.int32)

    g = -jnp.log(-jnp.log(u_ref[...]))
    x = logits_ref[...] + g
    cols = j * BCA + lax.broadcasted_iota(jnp.int32, (B, BCA), 1)
    x = jnp.where(cols < V, x, -jnp.inf)
    bmax = jnp.max(x, axis=1, keepdims=True)
    barg = jnp.argmax(x, axis=1).astype(jnp.int32)[:, None] + j * BCA
    upd = bmax > best_ref[...]
    best_ref[...] = jnp.where(upd, bmax, best_ref[...])
    bidx_ref[...] = jnp.where(upd, barg, bidx_ref[...])
    idx_ref[...] = bidx_ref[...]


def _onehot_body(idx_ref, out_ref):
    j = pl.program_id(0)
    cols = j * BCW + lax.broadcasted_iota(jnp.int32, (B, BCW), 1)
    out_ref[...] = (cols == idx_ref[...]).astype(jnp.float32)


@jax.jit
def _run(logits, u):
    idx = pl.pallas_call(
        _argmax_body,
        grid=(NBA,),
        in_specs=[
            pl.BlockSpec((B, BCA), lambda j: (0, j)),
            pl.BlockSpec((B, BCA), lambda j: (0, j)),
        ],
        out_specs=pl.BlockSpec((B, 1), lambda j: (0, 0)),
        out_shape=jax.ShapeDtypeStruct((B, 1), jnp.int32),
        scratch_shapes=[
            pltpu.VMEM((B, 1), jnp.float32),
            pltpu.VMEM((B, 1), jnp.int32),
        ],
    )(logits, u)
    onehot = pl.pallas_call(
        _onehot_body,
        grid=(NBW,),
        in_specs=[pl.BlockSpec((B, 1), lambda j: (0, 0))],
        out_specs=pl.BlockSpec((B, BCW), lambda j: (0, j)),
        out_shape=jax.ShapeDtypeStruct((B, V), jnp.float32),
    )(idx)
    return onehot


def kernel(logits):
    return _run(logits, jnp.asarray(_U))


# speedup vs baseline: 1.0207x; 1.0207x over previous
import jax
import jax.numpy as jnp
import numpy as np
from jax import lax
from jax.experimental import pallas as pl
from jax.experimental.pallas import tpu as pltpu

B = 128
V = 100000
BCA = 8192
NBA = (V + BCA - 1) // BCA
BCW = 16384
NBW = (V + BCW - 1) // BCW


def _threefry2x32(k0, k1, x0, x1):
    rotations = ((13, 15, 26, 6), (17, 29, 16, 24))
    ks = (np.uint32(k0), np.uint32(k1),
          np.uint32(k0) ^ np.uint32(k1) ^ np.uint32(0x1BD11BDA))
    x0 = (x0 + ks[0]).astype(np.uint32)
    x1 = (x1 + ks[1]).astype(np.uint32)
    for i in range(5):
        for r in rotations[i % 2]:
            x0 = (x0 + x1).astype(np.uint32)
            x1 = ((x1 << np.uint32(r)) | (x1 >> np.uint32(32 - r))).astype(np.uint32)
            x1 = x1 ^ x0
        x0 = (x0 + ks[(i + 1) % 3]).astype(np.uint32)
        x1 = (x1 + ks[(i + 2) % 3] + np.uint32(i + 1)).astype(np.uint32)
    return x0, x1


def _uniform_const():
    idx = np.arange(B * V, dtype=np.uint64)
    b0, b1 = _threefry2x32(0, 42,
                           (idx >> np.uint64(32)).astype(np.uint32),
                           (idx & np.uint64(0xFFFFFFFF)).astype(np.uint32))
    bits = b0 ^ b1
    fl = ((bits >> np.uint32(9)) | np.uint32(0x3F800000)).view(np.float32)
    fl = fl - np.float32(1.0)
    tiny = np.float32(np.finfo(np.float32).tiny)
    u = np.maximum(tiny, fl * (np.float32(1.0) - tiny) + tiny)
    return u.reshape(B, V)


_U = _uniform_const()


def _argmax_body(logits_ref, u_ref, idx_ref, best_ref, bidx_ref):
    j = pl.program_id(0)

    @pl.when(j == 0)
    def _():
        best_ref[...] = jnp.full((B, 1), -jnp.inf, jnp.float32)
        bidx_ref[...] = jnp.zeros((B, 1), jnp.int32)

    g = -jnp.log(-jnp.log(u_ref[...]))
    x = logits_ref[...] + g
    cols = j * BCA + lax.broadcasted_iota(jnp.int32, (B, BCA), 1)
    x = jnp.where(cols < V, x, -jnp.inf)
    bmax = jnp.max(x, axis=1, keepdims=True)
    barg = jnp.argmax(x, axis=1).astype(jnp.int32)[:, None] + j * BCA
    upd = bmax > best_ref[...]
    best_ref[...] = jnp.where(upd, bmax, best_ref[...])
    bidx_ref[...] = jnp.where(upd, barg, bidx_ref[...])
    idx_ref[...] = bidx_ref[...]


def _onehot_body(idx_ref, out_ref):
    j = pl.program_id(0)
    cols = j * BCW + lax.broadcasted_iota(jnp.int32, (B, BCW), 1)
    out_ref[...] = (cols == idx_ref[...]).astype(jnp.float32)


@jax.jit
def _run(logits, u):
    idx = pl.pallas_call(
        _argmax_body,
        grid=(NBA,),
        in_specs=[
            pl.BlockSpec((B, BCA), lambda j: (0, j)),
            pl.BlockSpec((B, BCA), lambda j: (0, j)),
        ],
        out_specs=pl.BlockSpec((B, 1), lambda j: (0, 0)),
        out_shape=jax.ShapeDtypeStruct((B, 1), jnp.int32),
        scratch_shapes=[
            pltpu.VMEM((B, 1), jnp.float32),
            pltpu.VMEM((B, 1), jnp.int32),
        ],
    )(logits, u)
    onehot = pl.pallas_call(
        _onehot_body,
        grid=(NBW,),
        in_specs=[pl.BlockSpec((B, 1), lambda j: (0, 0))],
        out_specs=pl.BlockSpec((B, BCW), lambda j: (0, j)),
        out_shape=jax.ShapeDtypeStruct((B, V), jnp.float32),
    )(idx)
    return onehot


def kernel(logits):
    return _run(logits, jnp.asarray(_U))


# speedup vs baseline: 1.0215x; 1.0008x over previous
import jax
import jax.numpy as jnp
import numpy as np
from jax import lax
from jax.experimental import pallas as pl
from jax.experimental.pallas import tpu as pltpu

B = 128
V = 100000
BCA = 8192
NBA = (V + BCA - 1) // BCA
BCW = 8192
NBW = (V + BCW - 1) // BCW


def _threefry2x32(k0, k1, x0, x1):
    rotations = ((13, 15, 26, 6), (17, 29, 16, 24))
    ks = (np.uint32(k0), np.uint32(k1),
          np.uint32(k0) ^ np.uint32(k1) ^ np.uint32(0x1BD11BDA))
    x0 = (x0 + ks[0]).astype(np.uint32)
    x1 = (x1 + ks[1]).astype(np.uint32)
    for i in range(5):
        for r in rotations[i % 2]:
            x0 = (x0 + x1).astype(np.uint32)
            x1 = ((x1 << np.uint32(r)) | (x1 >> np.uint32(32 - r))).astype(np.uint32)
            x1 = x1 ^ x0
        x0 = (x0 + ks[(i + 1) % 3]).astype(np.uint32)
        x1 = (x1 + ks[(i + 2) % 3] + np.uint32(i + 1)).astype(np.uint32)
    return x0, x1


def _uniform_const():
    idx = np.arange(B * V, dtype=np.uint64)
    b0, b1 = _threefry2x32(0, 42,
                           (idx >> np.uint64(32)).astype(np.uint32),
                           (idx & np.uint64(0xFFFFFFFF)).astype(np.uint32))
    bits = b0 ^ b1
    fl = ((bits >> np.uint32(9)) | np.uint32(0x3F800000)).view(np.float32)
    fl = fl - np.float32(1.0)
    tiny = np.float32(np.finfo(np.float32).tiny)
    u = np.maximum(tiny, fl * (np.float32(1.0) - tiny) + tiny)
    return u.reshape(B, V)


_U = _uniform_const()


def _argmax_body(logits_ref, u_ref, idx_ref, best_ref, bidx_ref):
    j = pl.program_id(0)

    @pl.when(j == 0)
    def _():
        best_ref[...] = jnp.full((B, 1), -jnp.inf, jnp.float32)
        bidx_ref[...] = jnp.zeros((B, 1), jnp.int32)

    g = -jnp.log(-jnp.log(u_ref[...]))
    x = logits_ref[...] + g
    cols = j * BCA + lax.broadcasted_iota(jnp.int32, (B, BCA), 1)
    x = jnp.where(cols < V, x, -jnp.inf)
    bmax = jnp.max(x, axis=1, keepdims=True)
    barg = jnp.argmax(x, axis=1).astype(jnp.int32)[:, None] + j * BCA
    upd = bmax > best_ref[...]
    best_ref[...] = jnp.where(upd, bmax, best_ref[...])
    bidx_ref[...] = jnp.where(upd, barg, bidx_ref[...])
    idx_ref[...] = bidx_ref[...]


def _onehot_body(idx_ref, out_ref):
    j = pl.program_id(0)
    cols = j * BCW + lax.broadcasted_iota(jnp.int32, (B, BCW), 1)
    out_ref[...] = (cols == idx_ref[...]).astype(jnp.float32)


@jax.jit
def _run(logits, u):
    idx = pl.pallas_call(
        _argmax_body,
        grid=(NBA,),
        in_specs=[
            pl.BlockSpec((B, BCA), lambda j: (0, j)),
            pl.BlockSpec((B, BCA), lambda j: (0, j)),
        ],
        out_specs=pl.BlockSpec((B, 1), lambda j: (0, 0)),
        out_shape=jax.ShapeDtypeStruct((B, 1), jnp.int32),
        scratch_shapes=[
            pltpu.VMEM((B, 1), jnp.float32),
            pltpu.VMEM((B, 1), jnp.int32),
        ],
    )(logits, u)
    onehot = pl.pallas_call(
        _onehot_body,
        grid=(NBW,),
        in_specs=[pl.BlockSpec((B, 1), lambda j: (0, 0))],
        out_specs=pl.BlockSpec((B, BCW), lambda j: (0, j)),
        out_shape=jax.ShapeDtypeStruct((B, V), jnp.float32),
    )(idx)
    return onehot


def kernel(logits):
    return _run(logits, jnp.asarray(_U))
